# baseline (speedup 1.0000x reference)
import functools

import jax
import jax.numpy as jnp
from jax import lax
from jax.experimental import pallas as pl
from jax.experimental.pallas import tpu as pltpu
from jax.experimental.pallas import tpu_sc as plsc

_D = 32
_B = 16384 * 20

_info = plsc.get_sparse_core_info()
_NC = _info.num_cores
_NS = _info.num_subcores
_NW = _NC * _NS
_BPW = _B // _NW
_C = 1024
_NCHUNK = _BPW // _C

_mesh = plsc.VectorSubcoreMesh(core_axis_name="c", subcore_axis_name="s")


@functools.partial(
    pl.kernel,
    mesh=_mesh,
    out_type=jax.ShapeDtypeStruct((_B, _D), jnp.float32),
    scratch_types=[
        pltpu.VMEM((_C,), jnp.int32),
        pltpu.VMEM((_C, _D), jnp.float32),
        pltpu.SemaphoreType.DMA,
    ],
    compiler_params=pltpu.CompilerParams(use_tc_tiling_on_sc=False),
)
def _gather_kernel(table_hbm, idx_hbm, out_hbm, idx_v, rows_v, sem):
    wid = lax.axis_index("s") * _NC + lax.axis_index("c")
    base = wid * _BPW

    def body(i, carry):
        off = base + i * _C
        pltpu.sync_copy(idx_hbm.at[pl.ds(off, _C)], idx_v)
        pltpu.async_copy(table_hbm.at[idx_v], rows_v, sem).wait()
        pltpu.sync_copy(rows_v, out_hbm.at[pl.ds(off, _C)])
        return carry

    lax.fori_loop(0, _NCHUNK, body, 0)


def kernel(arg1_1, arg223_1):
    idx = arg223_1.reshape(-1).astype(jnp.int32)
    return _gather_kernel(arg1_1, idx)


# speedup vs baseline: 1.0152x; 1.0152x over previous
import functools

import jax
import jax.numpy as jnp
from jax import lax
from jax.experimental import pallas as pl
from jax.experimental.pallas import tpu as pltpu
from jax.experimental.pallas import tpu_sc as plsc

_D = 32
_B = 16384 * 20

_info = plsc.get_sparse_core_info()
_NC = _info.num_cores
_NS = _info.num_subcores
_NW = _NC * _NS
_BPW = _B // _NW
_C = 1024
_NCHUNK = _BPW // _C
_NBUF = 3

_mesh = plsc.VectorSubcoreMesh(core_axis_name="c", subcore_axis_name="s")


@functools.partial(
    pl.kernel,
    mesh=_mesh,
    out_type=jax.ShapeDtypeStruct((_B, _D), jnp.float32),
    scratch_types=[
        pltpu.VMEM((_BPW,), jnp.int32),
        [pltpu.VMEM((_C, _D), jnp.float32) for _ in range(_NBUF)],
        [pltpu.SemaphoreType.DMA for _ in range(_NBUF)],
        [pltpu.SemaphoreType.DMA for _ in range(_NBUF)],
    ],
    compiler_params=pltpu.CompilerParams(use_tc_tiling_on_sc=False),
)
def _gather_kernel(table_hbm, idx_hbm, out_hbm, idx_v, rows, gsem, wsem):
    wid = lax.axis_index("s") * _NC + lax.axis_index("c")
    base = wid * _BPW

    pltpu.sync_copy(idx_hbm.at[pl.ds(base, _BPW)], idx_v)

    def fire_gather(i, b):
        pltpu.async_copy(
            table_hbm.at[idx_v.at[pl.ds(i * _C, _C)]], rows[b], gsem[b]
        )

    for i in range(_NBUF):
        fire_gather(i, i)

    for i in range(_NCHUNK):
        b = i % _NBUF
        pltpu.make_async_copy(
            table_hbm.at[idx_v.at[pl.ds(i * _C, _C)]], rows[b], gsem[b]
        ).wait()
        wcopy = pltpu.make_async_copy(
            rows[b], out_hbm.at[pl.ds(base + i * _C, _C)], wsem[b]
        )
        wcopy.start()
        ni = i + _NBUF
        if ni < _NCHUNK:
            wcopy.wait()
            fire_gather(ni, b)

    for j in range(max(0, _NCHUNK - _NBUF), _NCHUNK):
        bb = j % _NBUF
        pltpu.make_async_copy(
            rows[bb], out_hbm.at[pl.ds(base + j * _C, _C)], wsem[bb]
        ).wait()


def kernel(arg1_1, arg223_1):
    idx = arg223_1.reshape(-1).astype(jnp.int32)
    return _gather_kernel(arg1_1, idx)
